# baseline (speedup 1.0000x reference)
import functools

import numpy as np
import jax
import jax.numpy as jnp
from jax import lax
from jax.experimental import pallas as pl
from jax.experimental.pallas import tpu as pltpu
from jax.experimental.pallas import tpu_sc as plsc

B, S, V, D = 1024, 50, 100000, 128
N = B * S

_NC, _NS = 2, 16
NW = _NC * _NS
PER_W = N // NW
CH = 128
NCH = (PER_W + CH - 1) // CH
TAIL = PER_W - (NCH - 1) * CH
IDX_PAD = NCH * CH


def _make_pe():
    position = np.arange(S)[:, None].astype(np.float64)
    div_term = np.exp(np.arange(0, D, 2).astype(np.float64) * (-np.log(10000.0) / D))
    pe = np.zeros((S, D), dtype=np.float32)
    pe[:, 0::2] = np.sin(position * div_term)
    pe[:, 1::2] = np.cos(position * div_term)
    return pe


def _sc_gather3(tok_tab, act_tab, res_tab, tok_idx, act_idx, res_idx):
    mesh = plsc.VectorSubcoreMesh(
        core_axis_name="c", subcore_axis_name="s",
        num_cores=_NC, num_subcores=_NS)

    @functools.partial(
        pl.kernel,
        out_type=(jax.ShapeDtypeStruct((N, D), jnp.float32),) * 3,
        mesh=mesh,
        scratch_types=[
            pltpu.VMEM((IDX_PAD,), jnp.int32),
            pltpu.VMEM((CH, D), jnp.float32),
            pltpu.VMEM((CH, D), jnp.float32),
            pltpu.SemaphoreType.DMA,
            pltpu.SemaphoreType.DMA,
        ],
    )
    def gather_k(tok_tab, act_tab, res_tab, tok_i, act_i, res_i,
                 o_tok, o_act, o_res, idx_v, buf_a, buf_b, sem_a, sem_b):
        wid = lax.axis_index("s") * _NC + lax.axis_index("c")
        base = wid * PER_W
        zeros16 = jnp.zeros((16,), jnp.int32)
        for tab, idx_hbm, out_hbm in ((tok_tab, tok_i, o_tok),
                                      (act_tab, act_i, o_act),
                                      (res_tab, res_i, o_res)):
            pltpu.sync_copy(idx_hbm.at[pl.ds(base, PER_W)],
                            idx_v.at[pl.ds(0, PER_W)])
            for i in range(PER_W, IDX_PAD, 16):
                idx_v[pl.ds(i, 16)] = zeros16

            def start(c, buf, sem, tab=tab):
                return pltpu.async_copy(
                    tab.at[idx_v.at[pl.ds(c * CH, CH)]], buf, sem)

            def store(c, buf, out_hbm=out_hbm):
                pltpu.sync_copy(buf, out_hbm.at[pl.ds(base + c * CH, CH)])

            start(0, buf_a, sem_a)

            def pair(c, carry, tab=tab, out_hbm=out_hbm):
                k0 = 2 * c
                pltpu.async_copy(
                    tab.at[idx_v.at[pl.ds((k0 + 1) * CH, CH)]], buf_b, sem_b)
                pltpu.make_async_copy(
                    tab.at[idx_v.at[pl.ds(k0 * CH, CH)]], buf_a, sem_a).wait()
                pltpu.sync_copy(buf_a, out_hbm.at[pl.ds(base + k0 * CH, CH)])
                pltpu.async_copy(
                    tab.at[idx_v.at[pl.ds((k0 + 2) * CH, CH)]], buf_a, sem_a)
                pltpu.make_async_copy(
                    tab.at[idx_v.at[pl.ds((k0 + 1) * CH, CH)]], buf_b, sem_b).wait()
                pltpu.sync_copy(buf_b, out_hbm.at[pl.ds(base + (k0 + 1) * CH, CH)])
                return carry

            lax.fori_loop(0, (NCH - 1) // 2, pair, 0)
            pltpu.make_async_copy(
                tab.at[idx_v.at[pl.ds((NCH - 1) * CH, CH)]], buf_a, sem_a).wait()
            pltpu.sync_copy(buf_a.at[pl.ds(0, TAIL)],
                            out_hbm.at[pl.ds(base + (NCH - 1) * CH, TAIL)])

    return gather_k(tok_tab, act_tab, res_tab, tok_idx, act_idx, res_idx)


def _fold_body(nW2, tW2, pW, nb2, tb2, cn_ref, ct_ref, bvec_ref):
    wn = pW[2 * D:3 * D, :]
    wt = pW[3 * D:4 * D, :]
    cn_ref[...] = jnp.dot(nW2[...], wn, preferred_element_type=jnp.float32)
    ct_ref[...] = jnp.dot(tW2[...], wt, preferred_element_type=jnp.float32)
    bvec_ref[...] = (jnp.dot(nb2[...], wn, preferred_element_type=jnp.float32)
                     + jnp.dot(tb2[...], wt, preferred_element_type=jnp.float32))


def _fold(num_W2, time_W2, proj_W, num_b2, time_b2):
    return pl.pallas_call(
        _fold_body,
        out_shape=(jax.ShapeDtypeStruct((D // 2, D), jnp.float32),
                   jax.ShapeDtypeStruct((D // 2, D), jnp.float32),
                   jax.ShapeDtypeStruct((1, D), jnp.float32)),
    )(num_W2, time_W2, proj_W, num_b2, time_b2)


NR = 1024


def _tc_body(aux_ref, tok_ref, act_ref, res_ref, nW1, nb1, tW1, tb1,
             pW2, cn, ct, bvec, pe_ref, out_ref):
    a = aux_ref[...]
    m = a[:, 0:1]
    nf = a[:, 1:2]
    t0 = a[:, 2:3]
    t1 = a[:, 3:4]
    h_n = jnp.maximum(nf * nW1[0:1, :] + nb1[0:1, :], 0.0)
    h_t = jnp.maximum(t0 * tW1[0:1, :] + t1 * tW1[1:2, :] + tb1[0:1, :], 0.0)
    w = pW2[...]
    p = (jnp.dot(act_ref[...], w[0:D, :], preferred_element_type=jnp.float32)
         + jnp.dot(res_ref[...], w[D:2 * D, :], preferred_element_type=jnp.float32)
         + jnp.dot(h_n, cn[...], preferred_element_type=jnp.float32)
         + jnp.dot(h_t, ct[...], preferred_element_type=jnp.float32)
         + bvec[0:1, :])
    out_ref[...] = m * p + tok_ref[...] + pe_ref[0]


def _tc_fuse(aux, tok_rows, act_rows, res_rows,
             num_W1, num_b1, time_W1, time_b1, proj_W, cn, ct, bvec, pe_pb):
    rows_spec = pl.BlockSpec((NR, D), lambda i: (i, 0))
    full = lambda shape: pl.BlockSpec(shape, lambda i: (0,) * len(shape))
    return pl.pallas_call(
        _tc_body,
        grid=(N // NR,),
        in_specs=[
            pl.BlockSpec((NR, 4), lambda i: (i, 0)),
            rows_spec, rows_spec, rows_spec,
            full((1, D // 2)), full((1, D // 2)),
            full((2, D // 2)), full((1, D // 2)),
            pl.BlockSpec((2 * D, D), lambda i: (0, 0)),
            full((D // 2, D)), full((D // 2, D)), full((1, D)),
            pl.BlockSpec((1, 1, D), lambda i: (i // (B // NR), 0, 0)),
        ],
        out_specs=pl.BlockSpec((NR, D), lambda i: (i, 0)),
        out_shape=jax.ShapeDtypeStruct((N, D), jnp.float32),
    )(aux, tok_rows, act_rows, res_rows,
      num_W1, num_b1, time_W1, time_b1, proj_W, cn, ct, bvec, pe_pb)


def kernel(token_ids, activity_ids, resource_ids, numeric_features, time_features,
           token_table, activity_table, resource_table,
           num_W1, num_b1, num_W2, num_b2,
           time_W1, time_b1, time_W2, time_b2,
           proj_W, proj_b):
    tok_idx = token_ids.T.reshape(N).astype(jnp.int32)
    act_idx = activity_ids.T.reshape(N).astype(jnp.int32)
    res_idx = resource_ids.T.reshape(N).astype(jnp.int32)
    mask = (activity_ids.T > 0).astype(jnp.float32)[..., None]
    numT = numeric_features.transpose(1, 0, 2)
    timeT = time_features.transpose(1, 0, 2)
    aux = jnp.concatenate([mask, numT, timeT], axis=-1).reshape(N, 4)

    tok_rows, act_rows, res_rows = _sc_gather3(
        token_table, activity_table, resource_table, tok_idx, act_idx, res_idx)

    cn, ct, bvec = _fold(num_W2, time_W2, proj_W,
                         num_b2.reshape(1, D), time_b2.reshape(1, D))
    pe_pb = (jnp.asarray(_make_pe()) + proj_b[None, :]).reshape(S, 1, D)
    out = _tc_fuse(aux, tok_rows, act_rows, res_rows,
                   num_W1.reshape(1, D // 2), num_b1.reshape(1, D // 2),
                   time_W1, time_b1.reshape(1, D // 2),
                   proj_W, cn, ct, bvec, pe_pb)
    return out.reshape(S, B, D)


# speedup vs baseline: 2.2913x; 2.2913x over previous
import functools

import numpy as np
import jax
import jax.numpy as jnp
from jax import lax
from jax.experimental import pallas as pl
from jax.experimental.pallas import tpu as pltpu
from jax.experimental.pallas import tpu_sc as plsc

B, S, V, D = 1024, 50, 100000, 128
N = B * S

_NC, _NS = 2, 16
NW = _NC * _NS
PER_W = N // NW
CH = 128
NCH = (PER_W + CH - 1) // CH
TAIL = PER_W - (NCH - 1) * CH
IDX_PAD = NCH * CH


def _make_pe():
    position = np.arange(S)[:, None].astype(np.float64)
    div_term = np.exp(np.arange(0, D, 2).astype(np.float64) * (-np.log(10000.0) / D))
    pe = np.zeros((S, D), dtype=np.float32)
    pe[:, 0::2] = np.sin(position * div_term)
    pe[:, 1::2] = np.cos(position * div_term)
    return pe


def _sc_gather3(tok_tab, act_tab, res_tab, tok_idx, act_idx, res_idx):
    mesh = plsc.VectorSubcoreMesh(
        core_axis_name="c", subcore_axis_name="s",
        num_cores=_NC, num_subcores=_NS)

    @functools.partial(
        pl.kernel,
        out_type=(jax.ShapeDtypeStruct((N, D), jnp.float32),) * 3,
        mesh=mesh,
        scratch_types=[
            pltpu.VMEM((PER_W,), jnp.int32),
            pltpu.VMEM((CH, D), jnp.float32),
            pltpu.SemaphoreType.DMA,
        ],
    )
    def gather_k(tok_tab, act_tab, res_tab, tok_i, act_i, res_i,
                 o_tok, o_act, o_res, idx_v, rows_v, sem):
        wid = lax.axis_index("s") * _NC + lax.axis_index("c")
        base = wid * PER_W
        for tab, idx_hbm, out_hbm in ((tok_tab, tok_i, o_tok),
                                      (act_tab, act_i, o_act),
                                      (res_tab, res_i, o_res)):
            pltpu.sync_copy(idx_hbm.at[pl.ds(base, PER_W)], idx_v)

            def chunk(c, carry, tab=tab, out_hbm=out_hbm):
                row0 = c * CH
                pltpu.async_copy(
                    tab.at[idx_v.at[pl.ds(row0, CH)]], rows_v, sem).wait()
                pltpu.sync_copy(rows_v, out_hbm.at[pl.ds(base + row0, CH)])
                return carry

            lax.fori_loop(0, NCH - 1, chunk, 0)
            row0 = (NCH - 1) * CH
            pltpu.async_copy(
                tab.at[idx_v.at[pl.ds(row0, TAIL)]],
                rows_v.at[pl.ds(0, TAIL)], sem).wait()
            pltpu.sync_copy(rows_v.at[pl.ds(0, TAIL)],
                            out_hbm.at[pl.ds(base + row0, TAIL)])

    return gather_k(tok_tab, act_tab, res_tab, tok_idx, act_idx, res_idx)


def _fold_body(nW2, tW2, pW, nb2, tb2, cn_ref, ct_ref, bvec_ref):
    wn = pW[2 * D:3 * D, :]
    wt = pW[3 * D:4 * D, :]
    cn_ref[...] = jnp.dot(nW2[...], wn, preferred_element_type=jnp.float32)
    ct_ref[...] = jnp.dot(tW2[...], wt, preferred_element_type=jnp.float32)
    bvec_ref[...] = (jnp.dot(nb2[...], wn, preferred_element_type=jnp.float32)
                     + jnp.dot(tb2[...], wt, preferred_element_type=jnp.float32))


def _fold(num_W2, time_W2, proj_W, num_b2, time_b2):
    return pl.pallas_call(
        _fold_body,
        out_shape=(jax.ShapeDtypeStruct((D // 2, D), jnp.float32),
                   jax.ShapeDtypeStruct((D // 2, D), jnp.float32),
                   jax.ShapeDtypeStruct((1, D), jnp.float32)),
    )(num_W2, time_W2, proj_W, num_b2, time_b2)


NR = 1024


def _tc_body(aux_ref, tok_ref, act_ref, res_ref, nW1, nb1, tW1, tb1,
             pW2, cn, ct, bvec, pe_ref, out_ref):
    a = aux_ref[...]
    m = a[:, 0:1]
    nf = a[:, 1:2]
    t0 = a[:, 2:3]
    t1 = a[:, 3:4]
    h_n = jnp.maximum(nf * nW1[0:1, :] + nb1[0:1, :], 0.0)
    h_t = jnp.maximum(t0 * tW1[0:1, :] + t1 * tW1[1:2, :] + tb1[0:1, :], 0.0)
    w = pW2[...]
    p = (jnp.dot(act_ref[...], w[0:D, :], preferred_element_type=jnp.float32)
         + jnp.dot(res_ref[...], w[D:2 * D, :], preferred_element_type=jnp.float32)
         + jnp.dot(h_n, cn[...], preferred_element_type=jnp.float32)
         + jnp.dot(h_t, ct[...], preferred_element_type=jnp.float32)
         + bvec[0:1, :])
    out_ref[...] = m * p + tok_ref[...] + pe_ref[0]


def _tc_fuse(aux, tok_rows, act_rows, res_rows,
             num_W1, num_b1, time_W1, time_b1, proj_W, cn, ct, bvec, pe_pb):
    rows_spec = pl.BlockSpec((NR, D), lambda i: (i, 0))
    full = lambda shape: pl.BlockSpec(shape, lambda i: (0,) * len(shape))
    return pl.pallas_call(
        _tc_body,
        grid=(N // NR,),
        in_specs=[
            pl.BlockSpec((NR, 4), lambda i: (i, 0)),
            rows_spec, rows_spec, rows_spec,
            full((1, D // 2)), full((1, D // 2)),
            full((2, D // 2)), full((1, D // 2)),
            pl.BlockSpec((2 * D, D), lambda i: (0, 0)),
            full((D // 2, D)), full((D // 2, D)), full((1, D)),
            pl.BlockSpec((1, 1, D), lambda i: (i // (B // NR), 0, 0)),
        ],
        out_specs=pl.BlockSpec((NR, D), lambda i: (i, 0)),
        out_shape=jax.ShapeDtypeStruct((N, D), jnp.float32),
    )(aux, tok_rows, act_rows, res_rows,
      num_W1, num_b1, time_W1, time_b1, proj_W, cn, ct, bvec, pe_pb)


def kernel(token_ids, activity_ids, resource_ids, numeric_features, time_features,
           token_table, activity_table, resource_table,
           num_W1, num_b1, num_W2, num_b2,
           time_W1, time_b1, time_W2, time_b2,
           proj_W, proj_b):
    tok_idx = token_ids.T.reshape(N).astype(jnp.int32)
    act_idx = activity_ids.T.reshape(N).astype(jnp.int32)
    res_idx = resource_ids.T.reshape(N).astype(jnp.int32)
    mask = (activity_ids.T > 0).astype(jnp.float32)[..., None]
    numT = numeric_features.transpose(1, 0, 2)
    timeT = time_features.transpose(1, 0, 2)
    aux = jnp.concatenate([mask, numT, timeT], axis=-1).reshape(N, 4)

    tok_rows, act_rows, res_rows = _sc_gather3(
        token_table, activity_table, resource_table, tok_idx, act_idx, res_idx)

    cn, ct, bvec = _fold(num_W2, time_W2, proj_W,
                         num_b2.reshape(1, D), time_b2.reshape(1, D))
    pe_pb = (jnp.asarray(_make_pe()) + proj_b[None, :]).reshape(S, 1, D)
    out = _tc_fuse(aux, tok_rows, act_rows, res_rows,
                   num_W1.reshape(1, D // 2), num_b1.reshape(1, D // 2),
                   time_W1, time_b1.reshape(1, D // 2),
                   proj_W, cn, ct, bvec, pe_pb)
    return out.reshape(S, B, D)
